# baseline (speedup 1.0000x reference)
import jax
import jax.numpy as jnp
from jax import lax
from jax.experimental import pallas as pl
from jax.experimental.pallas import tpu as pltpu, tpu_sc as plsc
import functools

VOCAB = 13
D = 128
L = 200
B = 4096
NC = 2
NS = 16
LANES = 16

ROWS = B * L
ROWS_SC = ROWS // 2
ROWS_TC = ROWS - ROWS_SC
ROWS_PER_W = ROWS_SC // (NC * NS)
CHUNK = 64
N_CHUNKS = ROWS_PER_W // CHUNK
NBUF = 8
LP = 208


def _body(x_hbm, tok_hbm, pos_hbm, out_hbm, *refs):
    f_sh, prow_v, xall, tokrep, idx16 = refs[:5]
    idxs = refs[5:5 + NBUF]
    rows = refs[5 + NBUF:5 + 2 * NBUF]
    xsem = refs[5 + 2 * NBUF]
    gsems = refs[6 + 2 * NBUF:6 + 3 * NBUF]
    ssems = refs[6 + 3 * NBUF:6 + 4 * NBUF]

    s = lax.axis_index("s")
    c = lax.axis_index("c")
    wid = c * NS + s
    wbase = wid * ROWS_PER_W
    iota = lax.iota(jnp.int32, LANES)

    xcopy = pltpu.async_copy(x_hbm.at[pl.ds(wbase, ROWS_PER_W)], xall, xsem)

    @pl.when(s < VOCAB)
    def _fill():
        vbase = s * LP
        pcopy = pltpu.async_copy(
            pos_hbm.at[pl.ds(0, LP)], f_sh.at[pl.ds(vbase, LP)], gsems[0])
        pltpu.sync_copy(tok_hbm.at[s], prow_v)

        def rep(i, _):
            for cc in range(D // LANES):
                sl = pl.ds(cc * LANES, LANES)
                tokrep[i, sl] = prow_v[sl]
            return 0

        lax.fori_loop(0, 64, rep, 0)
        for r in range(3):
            for k in range(4):
                idxs[r][pl.ds(k * LANES, LANES)] = (
                    vbase + r * 64 + k * LANES + iota)
        idx16[pl.ds(0, LANES)] = vbase + 192 + iota
        pcopy.wait()
        for r in range(3):
            pltpu.sync_copy(tokrep, f_sh.at[idxs[r]], add=True)
        pltpu.sync_copy(tokrep.at[pl.ds(0, LANES)], f_sh.at[idx16], add=True)

    xcopy.wait()
    plsc.subcore_barrier()

    def start_g(b, t):
        for cc in range(CHUNK // LANES):
            off = t * CHUNK + cc * LANES
            p16 = lax.rem(iota + (wbase + off), L)
            idxs[b][pl.ds(cc * LANES, LANES)] = (
                xall[pl.ds(off, LANES)] * LP + p16)
        pltpu.async_copy(f_sh.at[idxs[b]], rows[b], gsems[b])

    def wait_g(b):
        pltpu.make_async_copy(f_sh.at[idxs[b]], rows[b], gsems[b]).wait()

    def start_s(b, t):
        base = wbase + t * CHUNK
        pltpu.async_copy(rows[b], out_hbm.at[pl.ds(base, CHUNK)], ssems[b])

    def wait_s(b):
        pltpu.make_async_copy(
            rows[b], out_hbm.at[pl.ds(wbase, CHUNK)], ssems[b]).wait()

    for b in range(NBUF):
        start_g(b, b)
    for b in range(NBUF - 1):
        wait_g(b)
        start_s(b, b)

    def outer(t0, _):
        for b in range(NBUF):
            t = t0 * NBUF + b
            wait_s(b)
            start_g(b, t)
            bp = (b - 1) % NBUF
            wait_g(bp)
            start_s(bp, t - 1)
        return 0

    lax.fori_loop(1, N_CHUNKS // NBUF, outer, 0)

    wait_g(NBUF - 1)
    start_s(NBUF - 1, N_CHUNKS - 1)
    for b in range(NBUF):
        wait_s(b)


@jax.jit
def _run(x_flat, token_table, position_embedding):
    mesh = plsc.VectorSubcoreMesh(
        core_axis_name="c", subcore_axis_name="s",
        num_cores=NC, num_subcores=NS)
    return pl.kernel(
        _body,
        out_type=jax.ShapeDtypeStruct((ROWS_SC, D), jnp.float32),
        mesh=mesh,
        scratch_types=[
            pltpu.VMEM_SHARED((LP * VOCAB, D), jnp.float32),
            pltpu.VMEM((D,), jnp.float32),
            pltpu.VMEM((ROWS_PER_W,), jnp.int32),
            pltpu.VMEM((64, D), jnp.float32),
            pltpu.VMEM((LANES,), jnp.int32),
            *[pltpu.VMEM((CHUNK,), jnp.int32) for _ in range(NBUF)],
            *[pltpu.VMEM((CHUNK, D), jnp.float32) for _ in range(NBUF)],
            pltpu.SemaphoreType.DMA,
            *[pltpu.SemaphoreType.DMA for _ in range(2 * NBUF)],
        ],
    )(x_flat, token_table, position_embedding)


TC_BLK_B = 16
TC_BLK_R = TC_BLK_B * L
TC_GRID = ROWS_TC // TC_BLK_R


def _tc_body(x_ref, tokp_ref, post_ref, o_ref):
    xb = x_ref[0, 0]
    oh = (xb[:, None] ==
          lax.broadcasted_iota(jnp.int32, (TC_BLK_R, 16), 1)
          ).astype(jnp.float32)
    o_ref[...] = jnp.dot(oh, tokp_ref[...],
                         preferred_element_type=jnp.float32) + post_ref[...]


@jax.jit
def _run_tc(x3, tokp, post):
    return pl.pallas_call(
        _tc_body,
        grid=(TC_GRID,),
        in_specs=[
            pl.BlockSpec((1, 1, TC_BLK_R), lambda j: (j, 0, 0)),
            pl.BlockSpec((16, D), lambda j: (0, 0)),
            pl.BlockSpec((TC_BLK_R, D), lambda j: (0, 0)),
        ],
        out_specs=pl.BlockSpec((TC_BLK_R, D), lambda j: (j, 0)),
        out_shape=jax.ShapeDtypeStruct((ROWS_TC, D), jnp.float32),
    )(x3, tokp, post)


def kernel(x, token_table, position_embedding):
    x_flat = x.reshape(-1).astype(jnp.int32)
    out_sc = _run(x_flat, token_table, position_embedding)
    x3 = x_flat[ROWS_SC:].reshape(TC_GRID, 1, TC_BLK_R)
    tokp = jnp.concatenate(
        [token_table, jnp.zeros((16 - VOCAB, D), jnp.float32)], axis=0)
    post = jnp.tile(position_embedding[:L], (TC_BLK_B, 1))
    out_tc = _run_tc(x3, tokp, post)
    out = jnp.concatenate([out_sc, out_tc], axis=0)
    return out.reshape(B, L, D)


# speedup vs baseline: 2.4398x; 2.4398x over previous
import jax
import jax.numpy as jnp
from jax import lax
from jax.experimental import pallas as pl
from jax.experimental.pallas import tpu as pltpu, tpu_sc as plsc

VOCAB = 13
D = 128
L = 200
B = 4096
NC = 2
NS = 16
LANES = 16

ROWS = B * L
ROWS_PER_W = ROWS // (NC * NS)
CHUNK = 64
N_CHUNKS = ROWS_PER_W // CHUNK
NBUF = 8
LP = 208


def _body(x_hbm, tok_hbm, pos_hbm, out_hbm, *refs):
    f_sh, prow_v, xall, tokrep, idx16 = refs[:5]
    idxs = refs[5:5 + NBUF]
    rows = refs[5 + NBUF:5 + 2 * NBUF]
    xsem = refs[5 + 2 * NBUF]
    gsems = refs[6 + 2 * NBUF:6 + 3 * NBUF]
    ssems = refs[6 + 3 * NBUF:6 + 4 * NBUF]

    s = lax.axis_index("s")
    c = lax.axis_index("c")
    wid = c * NS + s
    wbase = wid * ROWS_PER_W
    iota = lax.iota(jnp.int32, LANES)

    xcopy = pltpu.async_copy(x_hbm.at[pl.ds(wbase, ROWS_PER_W)], xall, xsem)

    @pl.when(s < VOCAB)
    def _fill():
        vbase = s * LP
        pcopy = pltpu.async_copy(
            pos_hbm.at[pl.ds(0, LP)], f_sh.at[pl.ds(vbase, LP)], gsems[0])
        pltpu.sync_copy(tok_hbm.at[s], prow_v)

        def rep(i, _):
            for cc in range(D // LANES):
                sl = pl.ds(cc * LANES, LANES)
                tokrep[i, sl] = prow_v[sl]
            return 0

        lax.fori_loop(0, 64, rep, 0)
        for r in range(3):
            for k in range(4):
                idxs[r][pl.ds(k * LANES, LANES)] = (
                    vbase + r * 64 + k * LANES + iota)
        idx16[pl.ds(0, LANES)] = vbase + 192 + iota
        pcopy.wait()
        for r in range(3):
            pltpu.sync_copy(tokrep, f_sh.at[idxs[r]], add=True)
        pltpu.sync_copy(tokrep.at[pl.ds(0, LANES)], f_sh.at[idx16], add=True)

    xcopy.wait()
    plsc.subcore_barrier()

    def start_g(b, t):
        for cc in range(CHUNK // LANES):
            off = t * CHUNK + cc * LANES
            p16 = lax.rem(iota + (wbase + off), L)
            idxs[b][pl.ds(cc * LANES, LANES)] = (
                xall[pl.ds(off, LANES)] * LP + p16)
        pltpu.async_copy(f_sh.at[idxs[b]], rows[b], gsems[b])

    def wait_g(b):
        pltpu.make_async_copy(f_sh.at[idxs[b]], rows[b], gsems[b]).wait()

    def start_s(b, t):
        base = wbase + t * CHUNK
        pltpu.async_copy(rows[b], out_hbm.at[pl.ds(base, CHUNK)], ssems[b])

    def wait_s(b):
        pltpu.make_async_copy(
            rows[b], out_hbm.at[pl.ds(wbase, CHUNK)], ssems[b]).wait()

    for b in range(NBUF):
        start_g(b, b)
    for b in range(NBUF - 1):
        wait_g(b)
        start_s(b, b)

    def outer(t0, _):
        for b in range(NBUF):
            t = t0 * NBUF + b
            wait_s(b)
            start_g(b, t)
            bp = (b - 1) % NBUF
            wait_g(bp)
            start_s(bp, t - 1)
        return 0

    lax.fori_loop(1, N_CHUNKS // NBUF, outer, 0)

    wait_g(NBUF - 1)
    start_s(NBUF - 1, N_CHUNKS - 1)
    for b in range(NBUF):
        wait_s(b)


@jax.jit
def _run(x_flat, token_table, position_embedding):
    mesh = plsc.VectorSubcoreMesh(
        core_axis_name="c", subcore_axis_name="s",
        num_cores=NC, num_subcores=NS)
    return pl.kernel(
        _body,
        out_type=jax.ShapeDtypeStruct((ROWS, D), jnp.float32),
        mesh=mesh,
        scratch_types=[
            pltpu.VMEM_SHARED((LP * VOCAB, D), jnp.float32),
            pltpu.VMEM((D,), jnp.float32),
            pltpu.VMEM((ROWS_PER_W,), jnp.int32),
            pltpu.VMEM((64, D), jnp.float32),
            pltpu.VMEM((LANES,), jnp.int32),
            *[pltpu.VMEM((CHUNK,), jnp.int32) for _ in range(NBUF)],
            *[pltpu.VMEM((CHUNK, D), jnp.float32) for _ in range(NBUF)],
            pltpu.SemaphoreType.DMA,
            *[pltpu.SemaphoreType.DMA for _ in range(2 * NBUF)],
        ],
    )(x_flat, token_table, position_embedding)


def kernel(x, token_table, position_embedding):
    x_flat = x.reshape(-1).astype(jnp.int32)
    out = _run(x_flat, token_table, position_embedding)
    return out.reshape(B, L, D)
